# initial kernel scaffold (unmeasured)
import jax
import jax.numpy as jnp
from jax import lax
from jax.experimental import pallas as pl
from jax.experimental.pallas import tpu as pltpu


def _allreduce_y(partial):
    t, d = partial.shape

    def body(p_ref, out_ref, comm_ref, send_sem, recv_sem):
        my_x = lax.axis_index("x")
        my_y = lax.axis_index("y")
        my_z = lax.axis_index("z")
        nbr = (my_x, 1 - my_y, my_z)

        barrier = pltpu.get_barrier_semaphore()
        pl.semaphore_signal(
            barrier, inc=1, device_id=nbr, device_id_type=pl.DeviceIdType.MESH
        )
        pl.semaphore_wait(barrier, 1)

        rdma = pltpu.make_async_remote_copy(
            src_ref=p_ref,
            dst_ref=comm_ref,
            send_sem=send_sem,
            recv_sem=recv_sem,
            device_id=nbr,
            device_id_type=pl.DeviceIdType.MESH,
        )
        rdma.start()
        rdma.wait()

        out_ref[...] = p_ref[...].astype(jnp.float32) + comm_ref[...].astype(
            jnp.float32
        )

    return pl.pallas_call(
        body,
        out_shape=jax.ShapeDtypeStruct((t, d), jnp.float32),
        in_specs=[pl.BlockSpec(memory_space=pltpu.VMEM)],
        out_specs=pl.BlockSpec(memory_space=pltpu.VMEM),
        scratch_shapes=[
            pltpu.VMEM((t, d), jnp.bfloat16),
            pltpu.SemaphoreType.DMA,
            pltpu.SemaphoreType.DMA,
        ],
        compiler_params=pltpu.CompilerParams(collective_id=0),
    )(partial)


def kernel(ids, E):
    v_per = E.shape[0]
    my_y = lax.axis_index("y")
    local = ids - my_y * v_per
    mask = (local >= 0) & (local < v_per)
    safe = jnp.where(mask, local, 0)
    partial = jnp.take(E, safe, axis=0)
    partial = jnp.where(mask[:, None], partial, 0.0).astype(jnp.bfloat16)
    return _allreduce_y(partial)


# baseline (device time: 376659 ns/iter reference)
import jax
import jax.numpy as jnp
from jax import lax
from jax.experimental import pallas as pl
from jax.experimental.pallas import tpu as pltpu


def _allreduce_y(partial):
    t, d = partial.shape

    def body(p_ref, out_ref, comm_ref, send_sem, recv_sem):
        my_x = lax.axis_index("x")
        my_y = lax.axis_index("y")
        my_z = lax.axis_index("z")
        nbr = (my_x, 1 - my_y, my_z)

        barrier = pltpu.get_barrier_semaphore()
        pl.semaphore_signal(
            barrier, inc=1, device_id=nbr, device_id_type=pl.DeviceIdType.MESH
        )
        pl.semaphore_wait(barrier, 1)

        rdma = pltpu.make_async_remote_copy(
            src_ref=p_ref,
            dst_ref=comm_ref,
            send_sem=send_sem,
            recv_sem=recv_sem,
            device_id=nbr,
            device_id_type=pl.DeviceIdType.MESH,
        )
        rdma.start()
        rdma.wait()

        out_ref[...] = p_ref[...] + comm_ref[...]

    return pl.pallas_call(
        body,
        out_shape=jax.ShapeDtypeStruct((t, d), jnp.bfloat16),
        in_specs=[pl.BlockSpec(memory_space=pltpu.VMEM)],
        out_specs=pl.BlockSpec(memory_space=pltpu.VMEM),
        input_output_aliases={0: 0},
        scratch_shapes=[
            pltpu.VMEM((t, d), jnp.bfloat16),
            pltpu.SemaphoreType.DMA,
            pltpu.SemaphoreType.DMA,
        ],
        compiler_params=pltpu.CompilerParams(
            collective_id=0, vmem_limit_bytes=56 * 1024 * 1024
        ),
    )(partial)


def kernel(ids, E):
    v_per = E.shape[0]
    my_y = lax.axis_index("y")
    local = ids - my_y * v_per
    mask = (local >= 0) & (local < v_per)
    safe = jnp.where(mask, local, 0)
    partial = jnp.take(E, safe, axis=0)
    partial = jnp.where(mask[:, None], partial, 0.0).astype(jnp.bfloat16)
    return _allreduce_y(partial).astype(jnp.float32)


# device time: 293838 ns/iter; 1.2819x vs baseline; 1.2819x over previous
import jax
import jax.numpy as jnp
from jax import lax
from jax.experimental import pallas as pl
from jax.experimental.pallas import tpu as pltpu


N_CHUNK = 16


def _allreduce_y(partial):
    t, d = partial.shape
    half = t // 2
    rows = half // N_CHUNK

    def body(p_ref, out_ref, ybuf, xbuf, ysend, yrecv, xsend, xrecv):
        my_x = lax.axis_index("x")
        my_y = lax.axis_index("y")
        my_z = lax.axis_index("z")
        ynbr = (my_x, 1 - my_y, my_z)
        xnbr = (1 - my_x, my_y, my_z)

        barrier = pltpu.get_barrier_semaphore()
        for nbr in (ynbr, xnbr):
            pl.semaphore_signal(
                barrier, inc=1, device_id=nbr, device_id_type=pl.DeviceIdType.MESH
            )
        pl.semaphore_wait(barrier, 2)

        my_off = my_x * half
        other_off = (1 - my_x) * half

        y_rdmas = []
        for c in range(N_CHUNK):
            r0 = c * rows
            rd = pltpu.make_async_remote_copy(
                src_ref=p_ref.at[pl.ds(my_off + r0, rows), :],
                dst_ref=ybuf.at[pl.ds(r0, rows), :],
                send_sem=ysend.at[c],
                recv_sem=yrecv.at[c],
                device_id=ynbr,
                device_id_type=pl.DeviceIdType.MESH,
            )
            rd.start()
            y_rdmas.append(rd)

        x_rdmas = []
        for c in range(N_CHUNK):
            r0 = c * rows
            y_rdmas[c].wait()
            rd = pltpu.make_async_remote_copy(
                src_ref=ybuf.at[pl.ds(r0, rows), :],
                dst_ref=xbuf.at[pl.ds(r0, rows), :],
                send_sem=xsend.at[c],
                recv_sem=xrecv.at[c],
                device_id=xnbr,
                device_id_type=pl.DeviceIdType.MESH,
            )
            rd.start()
            x_rdmas.append(rd)
            out_ref[pl.ds(my_off + r0, rows), :] = (
                p_ref[pl.ds(my_off + r0, rows), :] + ybuf[pl.ds(r0, rows), :]
            )

        for c in range(N_CHUNK):
            r0 = c * rows
            x_rdmas[c].wait_recv()
            out_ref[pl.ds(other_off + r0, rows), :] = (
                p_ref[pl.ds(other_off + r0, rows), :] + xbuf[pl.ds(r0, rows), :]
            )

        for c in range(N_CHUNK):
            x_rdmas[c].wait_send()

    return pl.pallas_call(
        body,
        out_shape=jax.ShapeDtypeStruct((t, d), jnp.bfloat16),
        in_specs=[pl.BlockSpec(memory_space=pltpu.VMEM)],
        out_specs=pl.BlockSpec(memory_space=pltpu.VMEM),
        input_output_aliases={0: 0},
        scratch_shapes=[
            pltpu.VMEM((half, d), jnp.bfloat16),
            pltpu.VMEM((half, d), jnp.bfloat16),
            pltpu.SemaphoreType.DMA((N_CHUNK,)),
            pltpu.SemaphoreType.DMA((N_CHUNK,)),
            pltpu.SemaphoreType.DMA((N_CHUNK,)),
            pltpu.SemaphoreType.DMA((N_CHUNK,)),
        ],
        compiler_params=pltpu.CompilerParams(
            collective_id=0, vmem_limit_bytes=56 * 1024 * 1024
        ),
    )(partial)


def kernel(ids, E):
    v_per = E.shape[0]
    my_y = lax.axis_index("y")
    local = ids - my_y * v_per
    mask = (local >= 0) & (local < v_per)
    safe = jnp.where(mask, local, 0)
    partial = jnp.take(E, safe, axis=0)
    partial = jnp.where(mask[:, None], partial, 0.0).astype(jnp.bfloat16)
    return _allreduce_y(partial).astype(jnp.float32)
